# baseline (speedup 1.0000x reference)
import functools

import jax
import jax.numpy as jnp
from jax.experimental import pallas as pl
from jax.experimental.pallas import tpu as pltpu

N, L, H = 1024, 50, 768
LN_EPS = 1e-12
BL = 2
K = 128


def _body(code_ref, table_ref, msum_ref, gamma_ref, beta_ref, out_ref):
    gamma = gamma_ref[...]
    beta = beta_ref[...]
    for ll in range(BL):
        code = code_ref[0, ll:ll + 1, :]
        tok = code & 31
        dec = (code >> 5) & 31
        chg = code >> 10
        dec = dec - 5 * (dec == 5).astype(dec.dtype)
        lp = BL * pl.program_id(0) + (ll + 50)
        row = jax.lax.broadcasted_iota(jnp.int32, (K, N), 0)
        hot = ((row == tok) | (row == dec + 30) | (row == chg + 40)
               | (row == lp))
        hot = hot.astype(jnp.float32)
        x = jax.lax.dot_general(
            hot, table_ref[...], (((0,), (0,)), ((), ())),
            preferred_element_type=jnp.float32)
        mm = jax.lax.dot_general(
            hot, msum_ref[...], (((0,), (0,)), ((), ())),
            preferred_element_type=jnp.float32)
        mean = mm[:, :1]
        ex2 = jnp.mean(x * x, axis=-1, keepdims=True)
        var = ex2 - mean * mean
        rstd = jax.lax.rsqrt(var + LN_EPS)
        y = (x * rstd - mean * rstd) * gamma + beta
        out_ref[ll] = y


@jax.jit
def kernel(peptide_tokens, decoration, charge, a_emb, charge_emb, phos_emb,
           pos_emb, ln_gamma, ln_beta):
    n, l = peptide_tokens.shape
    h = a_emb.shape[1]
    code = (peptide_tokens.T.astype(jnp.int32)
            | (decoration.T.astype(jnp.int32) << 5)
            | (charge.astype(jnp.int32)[None, :] << 10))
    code = code.reshape(l // BL, BL, n)
    table = jnp.concatenate(
        [a_emb, phos_emb, charge_emb, pos_emb[:l],
         jnp.zeros((K - 100, h), jnp.float32)], axis=0)
    msum = jnp.concatenate(
        [jnp.sum(table, axis=1, keepdims=True) / h,
         jnp.zeros((K, 127), jnp.float32)], axis=1)
    gamma = ln_gamma.reshape(1, h)
    beta = ln_beta.reshape(1, h)

    grid = (l // BL,)
    out = pl.pallas_call(
        _body,
        grid=grid,
        in_specs=[
            pl.BlockSpec((1, BL, n), lambda i: (i, 0, 0)),
            pl.BlockSpec((K, h), lambda i: (0, 0)),
            pl.BlockSpec((K, K), lambda i: (0, 0)),
            pl.BlockSpec((1, h), lambda i: (0, 0)),
            pl.BlockSpec((1, h), lambda i: (0, 0)),
        ],
        out_specs=pl.BlockSpec((BL, n, h), lambda i: (i, 0, 0)),
        out_shape=jax.ShapeDtypeStruct((l, n, h), jnp.float32),
        compiler_params=pltpu.CompilerParams(
            dimension_semantics=("arbitrary",),
        ),
    )(code, table, msum, gamma, beta)
    return out.transpose(1, 0, 2)


# speedup vs baseline: 1.0174x; 1.0174x over previous
import functools

import jax
import jax.numpy as jnp
from jax.experimental import pallas as pl
from jax.experimental.pallas import tpu as pltpu

N, L, H = 1024, 50, 768
LN_EPS = 1e-12
BN = 32
BT = BN * L
K = 128


def _body(code_ref, table_ref, msum_ref, gamma_ref, beta_ref, out_ref):
    code = code_ref[0]
    tok = code & 31
    dec = (code >> 5) & 31
    chg = code >> 10
    dec = dec - 5 * (dec == 5).astype(dec.dtype)
    lp = jax.lax.broadcasted_iota(jnp.int32, (1, BT), 1) // BN + 50
    row = jax.lax.broadcasted_iota(jnp.int32, (K, BT), 0)
    hot = ((row == tok) | (row == dec + 30) | (row == chg + 40)
           | (row == lp))
    hot = hot.astype(jnp.float32)
    x = jax.lax.dot_general(
        hot, table_ref[...], (((0,), (0,)), ((), ())),
        preferred_element_type=jnp.float32)
    mm = jax.lax.dot_general(
        hot, msum_ref[...], (((0,), (0,)), ((), ())),
        preferred_element_type=jnp.float32)
    mean = mm[:, :1]
    ex2 = jnp.mean(x * x, axis=-1, keepdims=True)
    var = ex2 - mean * mean
    rstd = jax.lax.rsqrt(var + LN_EPS)
    y = (x * rstd - mean * rstd) * gamma_ref[...] + beta_ref[...]
    out_ref[...] = y.reshape(L, BN, H)


@jax.jit
def kernel(peptide_tokens, decoration, charge, a_emb, charge_emb, phos_emb,
           pos_emb, ln_gamma, ln_beta):
    n, l = peptide_tokens.shape
    h = a_emb.shape[1]
    code = (peptide_tokens.astype(jnp.int32)
            | (decoration.astype(jnp.int32) << 5)
            | (charge.astype(jnp.int32)[:, None] << 10))
    code = (code.T.reshape(l, n // BN, BN).transpose(1, 0, 2)
            .reshape(n // BN, 1, BT))
    table = jnp.concatenate(
        [a_emb, phos_emb, charge_emb, pos_emb[:l],
         jnp.zeros((K - 100, h), jnp.float32)], axis=0)
    msum = jnp.concatenate(
        [jnp.sum(table, axis=1, keepdims=True) / h,
         jnp.zeros((K, 127), jnp.float32)], axis=1)
    gamma = ln_gamma.reshape(1, h)
    beta = ln_beta.reshape(1, h)

    grid = (n // BN,)
    out = pl.pallas_call(
        _body,
        grid=grid,
        in_specs=[
            pl.BlockSpec((1, 1, BT), lambda i: (i, 0, 0)),
            pl.BlockSpec((K, h), lambda i: (0, 0)),
            pl.BlockSpec((K, K), lambda i: (0, 0)),
            pl.BlockSpec((1, h), lambda i: (0, 0)),
            pl.BlockSpec((1, h), lambda i: (0, 0)),
        ],
        out_specs=pl.BlockSpec((l, BN, h), lambda i: (0, i, 0)),
        out_shape=jax.ShapeDtypeStruct((l, n, h), jnp.float32),
        compiler_params=pltpu.CompilerParams(
            dimension_semantics=("arbitrary",),
        ),
    )(code, table, msum, gamma, beta)
    return out.transpose(1, 0, 2)
